# baseline (speedup 1.0000x reference)
import functools

import jax
import jax.numpy as jnp
from jax import lax
from jax.experimental import pallas as pl
from jax.experimental.pallas import tpu as pltpu
from jax.experimental.pallas import tpu_sc as plsc

N = 10000
E = 320000
D = 128
NP = 10240
NC, NS, L = 2, 16, 16
NW = NC * NS
ROWS_PER_TILE = NP // NS
EDGE_CHUNK = 125
NCHUNKS = E // NW // EDGE_CHUNK
GROUP = 8
NGRP = NCHUNKS // GROUP
EROWS = 2 * E // EDGE_CHUNK
DROWS = 16
DCHUNKS = NCHUNKS // DROWS


def _mesh():
    return plsc.VectorSubcoreMesh(
        core_axis_name="c", subcore_axis_name="s",
        num_cores=NC, num_subcores=NS)


DTILE_ROWS = 2 * NCHUNKS
DNCHUNK = DTILE_ROWS // DROWS


def _degree_body(edge_hbm, dinv_hbm, idx_v, deg_v, acc_v, part_v, dv_v, shared,
                 semi0, semi1):
    c = lax.axis_index("c")
    s = lax.axis_index("s")
    dst_base = EROWS // 2 + s * DTILE_ROWS
    zero16 = jnp.zeros((L,), jnp.float32)
    ones16 = jnp.ones((L,), jnp.float32)
    tail_mask = lax.iota(jnp.int32, L) >= 3

    def zero_body(i, carry):
        deg_v[pl.ds(i * L, L)] = zero16
        return carry
    lax.fori_loop(0, NP // L, zero_body, 0)

    semi = (semi0, semi1)
    pend = [None, None]
    pend[0] = pltpu.async_copy(
        edge_hbm.at[pl.ds(dst_base, DROWS)], idx_v.at[0], semi[0])
    for k in range(DNCHUNK):
        b = k % 2
        if k + 1 < DNCHUNK:
            pend[1 - b] = pltpu.async_copy(
                edge_hbm.at[pl.ds(dst_base + (k + 1) * DROWS, DROWS)],
                idx_v.at[1 - b], semi[1 - b])
        pend[b].wait()
        for r in range(DROWS):
            for q in range(7):
                idx16 = idx_v[b, r, pl.ds(q * L, L)]
                plsc.addupdate_scatter(deg_v, [idx16], ones16)
            idx16 = idx_v[b, r, pl.ds(109, L)]
            plsc.addupdate_scatter(deg_v, [idx16], ones16, mask=tail_mask)

    pltpu.sync_copy(deg_v, shared.at[s])
    plsc.subcore_barrier()
    pltpu.sync_copy(shared.at[0, pl.ds(s * ROWS_PER_TILE, ROWS_PER_TILE)], acc_v)
    for p in range(1, NS):
        pltpu.sync_copy(shared.at[p, pl.ds(s * ROWS_PER_TILE, ROWS_PER_TILE)],
                        part_v)

        def add_body(i, carry):
            for u in range(4):
                o = (i * 4 + u) * L
                acc_v[pl.ds(o, L)] = acc_v[pl.ds(o, L)] + part_v[pl.ds(o, L)]
            return carry
        lax.fori_loop(0, ROWS_PER_TILE // L // 4, add_body, 0)

    def rsqrt_body(i, carry):
        d = acc_v[pl.ds(i * L, L)] + 1.0
        bits = plsc.bitcast(d, jnp.int32)
        bits = 0x5F3759DF - (bits >> 1)
        y = plsc.bitcast(bits, jnp.float32)
        for _ in range(3):
            y = y * (1.5 - 0.5 * d * y * y)
        dv_v[pl.ds(i * L, L)] = y
        return carry
    lax.fori_loop(0, ROWS_PER_TILE // L, rsqrt_body, 0)

    @pl.when(c == 0)
    def _():
        pltpu.sync_copy(
            dv_v, dinv_hbm.at[pl.ds(s * ROWS_PER_TILE, ROWS_PER_TILE)])


@functools.cache
def _degree_kernel_fn():
    return pl.kernel(
        _degree_body,
        out_type=jax.ShapeDtypeStruct((NP,), jnp.float32),
        mesh=_mesh(),
        compiler_params=pltpu.CompilerParams(needs_layout_passes=False),
        scratch_types=[
            pltpu.VMEM((2, DROWS, EDGE_CHUNK), jnp.int32),
            pltpu.VMEM((NP,), jnp.float32),
            pltpu.VMEM((ROWS_PER_TILE,), jnp.float32),
            pltpu.VMEM((ROWS_PER_TILE,), jnp.float32),
            pltpu.VMEM((ROWS_PER_TILE,), jnp.float32),
            pltpu.VMEM_SHARED((NS, NP), jnp.float32),
            pltpu.SemaphoreType.DMA,
            pltpu.SemaphoreType.DMA,
        ],
    )


def _degree_kernel(edge2d):
    return _degree_kernel_fn()(edge2d)


def _aggregate_body(edge_hbm, g_hbm, acc_hbm,
                    src_v, dst_v, rows0, rows1, acc_sh,
                    semg0, semg1, sems0, sems1, semi0, semi1, semi2):
    c = lax.axis_index("c")
    s = lax.axis_index("s")
    w = s * NC + c
    base = w * NCHUNKS

    pltpu.sync_copy(g_hbm.at[pl.ds(s * ROWS_PER_TILE, ROWS_PER_TILE)],
                    acc_sh.at[pl.ds(s * ROWS_PER_TILE, ROWS_PER_TILE)])

    pltpu.sync_copy(edge_hbm.at[pl.ds(base, GROUP)], src_v.at[0])
    pltpu.sync_copy(edge_hbm.at[pl.ds(EROWS // 2 + base, GROUP)], dst_v.at[0])
    plsc.subcore_barrier()

    rows = (rows0, rows1)
    semg = (semg0, semg1)
    sems = (sems0, sems1)
    semi = (semi0, semi1, semi2)
    pendg = [None, None]
    pends = [None, None]
    pendi = [None, None]
    SUB = 5
    SUBR = EDGE_CHUNK // SUB

    def _issue_gather(gbuf, jrow, buf):
        return [pltpu.async_copy(
                    g_hbm.at[src_v.at[gbuf, jrow, pl.ds(o * SUBR, SUBR)]],
                    rows[buf].at[pl.ds(o * SUBR, SUBR)], semg[buf])
                for o in range(SUB)]

    pendg[0] = _issue_gather(0, 0, 0)
    for grp in range(NGRP):
        gb = grp % 3
        if grp + 1 < NGRP:
            nb = (grp + 1) % 3
            nbase = base + (grp + 1) * GROUP
            pendi[0] = pltpu.async_copy(
                edge_hbm.at[pl.ds(nbase, GROUP)], src_v.at[nb], semi[nb])
            pendi[1] = pltpu.async_copy(
                edge_hbm.at[pl.ds(EROWS // 2 + nbase, GROUP)], dst_v.at[nb],
                semi[nb])
        for j in range(GROUP):
            k = grp * GROUP + j
            b = k % 2
            if k + 1 < NCHUNKS:
                g2 = (k + 1) // GROUP
                if g2 != grp:
                    pendi[0].wait()
                    pendi[1].wait()
                if pends[1 - b] is not None:
                    pends[1 - b].wait()
                    pends[1 - b] = None
                pendg[1 - b] = _issue_gather(g2 % 3, (k + 1) % GROUP, 1 - b)
            for dsc in pendg[b]:
                dsc.wait()
            pends[b] = pltpu.async_copy(
                rows[b], acc_sh.at[dst_v.at[gb, j]], sems[b], add=True)
    for b in range(2):
        if pends[b] is not None:
            pends[b].wait()

    plsc.subcore_barrier()
    pltpu.sync_copy(
        acc_sh.at[pl.ds(s * ROWS_PER_TILE, ROWS_PER_TILE)],
        acc_hbm.at[pl.ds(c * NP + s * ROWS_PER_TILE, ROWS_PER_TILE)])


@functools.cache
def _aggregate_kernel_fn():
    return pl.kernel(
        _aggregate_body,
        out_type=jax.ShapeDtypeStruct((NC * NP, D), jnp.float32),
        mesh=_mesh(),
        compiler_params=pltpu.CompilerParams(needs_layout_passes=False),
        scratch_types=[
            pltpu.VMEM((3, GROUP, EDGE_CHUNK), jnp.int32),
            pltpu.VMEM((3, GROUP, EDGE_CHUNK), jnp.int32),
            pltpu.VMEM((EDGE_CHUNK, D), jnp.float32),
            pltpu.VMEM((EDGE_CHUNK, D), jnp.float32),
            pltpu.VMEM_SHARED((NP, D), jnp.float32),
            pltpu.SemaphoreType.DMA,
            pltpu.SemaphoreType.DMA,
            pltpu.SemaphoreType.DMA,
            pltpu.SemaphoreType.DMA,
            pltpu.SemaphoreType.DMA,
            pltpu.SemaphoreType.DMA,
            pltpu.SemaphoreType.DMA,
        ],
    )


def _aggregate_kernel(edge2d, g):
    return _aggregate_kernel_fn()(edge2d, g)


RB = 1000


def _mlp_body(x_ref, dinv_ref, wi_ref, bi_ref, wh_ref, bh_ref, wg_ref,
              feat_ref, g_ref):
    t = jnp.dot(x_ref[...], wi_ref[...], preferred_element_type=jnp.float32)
    t = jnp.maximum(t + bi_ref[...], 0.0)
    f = jnp.dot(t, wh_ref[...], preferred_element_type=jnp.float32)
    f = jnp.maximum(f + bh_ref[...], 0.0)
    feat_ref[...] = f
    h = jnp.dot(f, wg_ref[...], preferred_element_type=jnp.float32)
    g_ref[...] = h * dinv_ref[...]


def _run_mlp(x, dinv2d, w_in, b_in, w_h, b_h, w_g):
    wspec = pl.BlockSpec((D, D), lambda i: (0, 0))
    bspec = pl.BlockSpec((1, D), lambda i: (0, 0))
    return pl.pallas_call(
        _mlp_body,
        grid=(N // RB,),
        in_specs=[
            pl.BlockSpec((RB, D), lambda i: (i, 0)),
            pl.BlockSpec((RB, 1), lambda i: (i, 0)),
            wspec, bspec, wspec, bspec, wspec,
        ],
        out_specs=[pl.BlockSpec((RB, D), lambda i: (i, 0)),
                   pl.BlockSpec((RB, D), lambda i: (i, 0))],
        out_shape=[jax.ShapeDtypeStruct((N, D), jnp.float32),
                   jax.ShapeDtypeStruct((NP, D), jnp.float32)],
    )(x, dinv2d, w_in, b_in, w_h, b_h, w_g)


def _final_body(acc_ref, g_ref, dinv_ref, bg_ref, out_ref):
    total = acc_ref[0] + acc_ref[1] - g_ref[...]
    out_ref[...] = total * dinv_ref[...] + bg_ref[...]


def _run_final(acc, g, dinv2d, b_g):
    return pl.pallas_call(
        _final_body,
        grid=(N // RB,),
        in_specs=[
            pl.BlockSpec((NC, RB, D), lambda i: (0, i, 0)),
            pl.BlockSpec((RB, D), lambda i: (i, 0)),
            pl.BlockSpec((RB, 1), lambda i: (i, 0)),
            pl.BlockSpec((1, D), lambda i: (0, 0)),
        ],
        out_specs=pl.BlockSpec((RB, D), lambda i: (i, 0)),
        out_shape=jax.ShapeDtypeStruct((N, D), jnp.float32),
    )(acc, g, dinv2d, b_g)


def kernel(x, edge_index, W_in, b_in, W_h, b_h, W_g, b_g):
    edge_index = edge_index.astype(jnp.int32)
    edge2d = edge_index.reshape(EROWS, EDGE_CHUNK)

    dinv2d = _degree_kernel(edge2d).reshape(NP, 1)

    feat, g = _run_mlp(x, dinv2d, W_in, b_in.reshape(1, D),
                       W_h, b_h.reshape(1, D), W_g)

    acc = _aggregate_kernel(edge2d, g)
    acc = acc.reshape(NC, NP, D)

    out_feat = _run_final(acc, g, dinv2d, b_g.reshape(1, D))
    return (feat, out_feat)


# speedup vs baseline: 1.0236x; 1.0236x over previous
import functools

import jax
import jax.numpy as jnp
from jax import lax
from jax.experimental import pallas as pl
from jax.experimental.pallas import tpu as pltpu
from jax.experimental.pallas import tpu_sc as plsc

N = 10000
E = 320000
D = 128
NP = 10240
NC, NS, L = 2, 16, 16
NW = NC * NS
ROWS_PER_TILE = NP // NS
EDGE_CHUNK = 125
NCHUNKS = E // NW // EDGE_CHUNK
GROUP = 8
NGRP = NCHUNKS // GROUP
EROWS = 2 * E // EDGE_CHUNK
DROWS = 16
DCHUNKS = NCHUNKS // DROWS


def _mesh():
    return plsc.VectorSubcoreMesh(
        core_axis_name="c", subcore_axis_name="s",
        num_cores=NC, num_subcores=NS)


DTILE_ROWS = 2 * NCHUNKS
DNCHUNK = DTILE_ROWS // DROWS


def _degree_body(edge_hbm, dinv_hbm, idx_v, deg_v, acc_v, part0_v, part1_v,
                 dv_v, shared, semi0, semi1):
    c = lax.axis_index("c")
    s = lax.axis_index("s")
    dst_base = EROWS // 2 + s * DTILE_ROWS
    zero16 = jnp.zeros((L,), jnp.float32)
    ones16 = jnp.ones((L,), jnp.float32)
    tail_mask = lax.iota(jnp.int32, L) >= 3

    def zero_body(i, carry):
        for u in range(8):
            deg_v[pl.ds((i * 8 + u) * L, L)] = zero16
        return carry
    lax.fori_loop(0, NP // L // 8, zero_body, 0)

    semi = (semi0, semi1)
    pend = [None, None]
    pend[0] = pltpu.async_copy(
        edge_hbm.at[pl.ds(dst_base, DROWS)], idx_v.at[0], semi[0])
    for k in range(DNCHUNK):
        b = k % 2
        if k + 1 < DNCHUNK:
            pend[1 - b] = pltpu.async_copy(
                edge_hbm.at[pl.ds(dst_base + (k + 1) * DROWS, DROWS)],
                idx_v.at[1 - b], semi[1 - b])
        pend[b].wait()
        for r in range(DROWS):
            for q in range(7):
                idx16 = idx_v[b, r, pl.ds(q * L, L)]
                plsc.addupdate_scatter(deg_v, [idx16], ones16)
            idx16 = idx_v[b, r, pl.ds(109, L)]
            plsc.addupdate_scatter(deg_v, [idx16], ones16, mask=tail_mask)

    pltpu.sync_copy(deg_v, shared.at[s])
    plsc.subcore_barrier()
    pltpu.sync_copy(shared.at[0, pl.ds(s * ROWS_PER_TILE, ROWS_PER_TILE)], acc_v)
    parts = (part0_v, part1_v)
    pendc = [None, None]
    pendc[1] = pltpu.async_copy(
        shared.at[1, pl.ds(s * ROWS_PER_TILE, ROWS_PER_TILE)], parts[1],
        semi0)
    for p in range(1, NS):
        pb = p % 2
        if p + 1 < NS:
            pendc[1 - pb] = pltpu.async_copy(
                shared.at[p + 1, pl.ds(s * ROWS_PER_TILE, ROWS_PER_TILE)],
                parts[1 - pb], semi1 if pb else semi0)
        pendc[pb].wait()

        def add_body(i, carry):
            for u in range(4):
                o = (i * 4 + u) * L
                acc_v[pl.ds(o, L)] = (acc_v[pl.ds(o, L)]
                                      + parts[pb][pl.ds(o, L)])
            return carry
        lax.fori_loop(0, ROWS_PER_TILE // L // 4, add_body, 0)

    def rsqrt_body(i, carry):
        d = acc_v[pl.ds(i * L, L)] + 1.0
        bits = plsc.bitcast(d, jnp.int32)
        bits = 0x5F3759DF - (bits >> 1)
        y = plsc.bitcast(bits, jnp.float32)
        for _ in range(3):
            y = y * (1.5 - 0.5 * d * y * y)
        dv_v[pl.ds(i * L, L)] = y
        return carry
    lax.fori_loop(0, ROWS_PER_TILE // L, rsqrt_body, 0)

    @pl.when(c == 0)
    def _():
        pltpu.sync_copy(
            dv_v, dinv_hbm.at[pl.ds(s * ROWS_PER_TILE, ROWS_PER_TILE)])


@functools.cache
def _degree_kernel_fn():
    return pl.kernel(
        _degree_body,
        out_type=jax.ShapeDtypeStruct((NP,), jnp.float32),
        mesh=_mesh(),
        compiler_params=pltpu.CompilerParams(needs_layout_passes=False),
        scratch_types=[
            pltpu.VMEM((2, DROWS, EDGE_CHUNK), jnp.int32),
            pltpu.VMEM((NP,), jnp.float32),
            pltpu.VMEM((ROWS_PER_TILE,), jnp.float32),
            pltpu.VMEM((ROWS_PER_TILE,), jnp.float32),
            pltpu.VMEM((ROWS_PER_TILE,), jnp.float32),
            pltpu.VMEM((ROWS_PER_TILE,), jnp.float32),
            pltpu.VMEM_SHARED((NS, NP), jnp.float32),
            pltpu.SemaphoreType.DMA,
            pltpu.SemaphoreType.DMA,
        ],
    )


def _degree_kernel(edge2d):
    return _degree_kernel_fn()(edge2d)


def _aggregate_body(edge_hbm, g_hbm, acc_hbm,
                    src_v, dst_v, rows0, rows1, acc_sh,
                    semg0, semg1, sems0, sems1, semi0, semi1, semi2):
    c = lax.axis_index("c")
    s = lax.axis_index("s")
    w = s * NC + c
    base = w * NCHUNKS

    pltpu.sync_copy(g_hbm.at[pl.ds(s * ROWS_PER_TILE, ROWS_PER_TILE)],
                    acc_sh.at[pl.ds(s * ROWS_PER_TILE, ROWS_PER_TILE)])

    pltpu.sync_copy(edge_hbm.at[pl.ds(base, GROUP)], src_v.at[0])
    pltpu.sync_copy(edge_hbm.at[pl.ds(EROWS // 2 + base, GROUP)], dst_v.at[0])
    plsc.subcore_barrier()

    rows = (rows0, rows1)
    semg = (semg0, semg1)
    sems = (sems0, sems1)
    semi = (semi0, semi1, semi2)
    pendg = [None, None]
    pends = [None, None]
    pendi = [None, None]
    pendg[0] = pltpu.async_copy(g_hbm.at[src_v.at[0, 0]], rows[0], semg[0])
    for grp in range(NGRP):
        gb = grp % 3
        if grp + 1 < NGRP:
            nb = (grp + 1) % 3
            nbase = base + (grp + 1) * GROUP
            pendi[0] = pltpu.async_copy(
                edge_hbm.at[pl.ds(nbase, GROUP)], src_v.at[nb], semi[nb])
            pendi[1] = pltpu.async_copy(
                edge_hbm.at[pl.ds(EROWS // 2 + nbase, GROUP)], dst_v.at[nb],
                semi[nb])
        for j in range(GROUP):
            k = grp * GROUP + j
            b = k % 2
            if k + 1 < NCHUNKS:
                g2 = (k + 1) // GROUP
                if g2 != grp:
                    pendi[0].wait()
                    pendi[1].wait()
                if pends[1 - b] is not None:
                    pends[1 - b].wait()
                    pends[1 - b] = None
                pendg[1 - b] = pltpu.async_copy(
                    g_hbm.at[src_v.at[g2 % 3, (k + 1) % GROUP]],
                    rows[1 - b], semg[1 - b])
            pendg[b].wait()
            pends[b] = pltpu.async_copy(
                rows[b], acc_sh.at[dst_v.at[gb, j]], sems[b], add=True)
    for b in range(2):
        if pends[b] is not None:
            pends[b].wait()

    plsc.subcore_barrier()
    pltpu.sync_copy(
        acc_sh.at[pl.ds(s * ROWS_PER_TILE, ROWS_PER_TILE)],
        acc_hbm.at[pl.ds(c * NP + s * ROWS_PER_TILE, ROWS_PER_TILE)])


@functools.cache
def _aggregate_kernel_fn():
    return pl.kernel(
        _aggregate_body,
        out_type=jax.ShapeDtypeStruct((NC * NP, D), jnp.float32),
        mesh=_mesh(),
        compiler_params=pltpu.CompilerParams(needs_layout_passes=False),
        scratch_types=[
            pltpu.VMEM((3, GROUP, EDGE_CHUNK), jnp.int32),
            pltpu.VMEM((3, GROUP, EDGE_CHUNK), jnp.int32),
            pltpu.VMEM((EDGE_CHUNK, D), jnp.float32),
            pltpu.VMEM((EDGE_CHUNK, D), jnp.float32),
            pltpu.VMEM_SHARED((NP, D), jnp.float32),
            pltpu.SemaphoreType.DMA,
            pltpu.SemaphoreType.DMA,
            pltpu.SemaphoreType.DMA,
            pltpu.SemaphoreType.DMA,
            pltpu.SemaphoreType.DMA,
            pltpu.SemaphoreType.DMA,
            pltpu.SemaphoreType.DMA,
        ],
    )


def _aggregate_kernel(edge2d, g):
    return _aggregate_kernel_fn()(edge2d, g)


RB = 1000


def _mlp_body(x_ref, dinv_ref, wi_ref, bi_ref, wh_ref, bh_ref, wg_ref,
              feat_ref, g_ref):
    t = jnp.dot(x_ref[...], wi_ref[...], preferred_element_type=jnp.float32)
    t = jnp.maximum(t + bi_ref[...], 0.0)
    f = jnp.dot(t, wh_ref[...], preferred_element_type=jnp.float32)
    f = jnp.maximum(f + bh_ref[...], 0.0)
    feat_ref[...] = f
    h = jnp.dot(f, wg_ref[...], preferred_element_type=jnp.float32)
    g_ref[...] = h * dinv_ref[...]


def _run_mlp(x, dinv2d, w_in, b_in, w_h, b_h, w_g):
    wspec = pl.BlockSpec((D, D), lambda i: (0, 0))
    bspec = pl.BlockSpec((1, D), lambda i: (0, 0))
    return pl.pallas_call(
        _mlp_body,
        grid=(N // RB,),
        in_specs=[
            pl.BlockSpec((RB, D), lambda i: (i, 0)),
            pl.BlockSpec((RB, 1), lambda i: (i, 0)),
            wspec, bspec, wspec, bspec, wspec,
        ],
        out_specs=[pl.BlockSpec((RB, D), lambda i: (i, 0)),
                   pl.BlockSpec((RB, D), lambda i: (i, 0))],
        out_shape=[jax.ShapeDtypeStruct((N, D), jnp.float32),
                   jax.ShapeDtypeStruct((NP, D), jnp.float32)],
    )(x, dinv2d, w_in, b_in, w_h, b_h, w_g)


def _final_body(acc_ref, g_ref, dinv_ref, bg_ref, out_ref):
    total = acc_ref[0] + acc_ref[1] - g_ref[...]
    out_ref[...] = total * dinv_ref[...] + bg_ref[...]


def _run_final(acc, g, dinv2d, b_g):
    return pl.pallas_call(
        _final_body,
        grid=(N // RB,),
        in_specs=[
            pl.BlockSpec((NC, RB, D), lambda i: (0, i, 0)),
            pl.BlockSpec((RB, D), lambda i: (i, 0)),
            pl.BlockSpec((RB, 1), lambda i: (i, 0)),
            pl.BlockSpec((1, D), lambda i: (0, 0)),
        ],
        out_specs=pl.BlockSpec((RB, D), lambda i: (i, 0)),
        out_shape=jax.ShapeDtypeStruct((N, D), jnp.float32),
    )(acc, g, dinv2d, b_g)


def kernel(x, edge_index, W_in, b_in, W_h, b_h, W_g, b_g):
    edge_index = edge_index.astype(jnp.int32)
    edge2d = edge_index.reshape(EROWS, EDGE_CHUNK)

    dinv2d = _degree_kernel(edge2d).reshape(NP, 1)

    feat, g = _run_mlp(x, dinv2d, W_in, b_in.reshape(1, D),
                       W_h, b_h.reshape(1, D), W_g)

    acc = _aggregate_kernel(edge2d, g)
    acc = acc.reshape(NC, NP, D)

    out_feat = _run_final(acc, g, dinv2d, b_g.reshape(1, D))
    return (feat, out_feat)


# speedup vs baseline: 1.0421x; 1.0180x over previous
import functools

import jax
import jax.numpy as jnp
from jax import lax
from jax.experimental import pallas as pl
from jax.experimental.pallas import tpu as pltpu
from jax.experimental.pallas import tpu_sc as plsc

N = 10000
E = 320000
D = 128
NP = 10240
NC, NS, L = 2, 16, 16
NW = NC * NS
ROWS_PER_TILE = NP // NS
EDGE_CHUNK = 125
NCHUNKS = E // NW // EDGE_CHUNK
GROUP = 8
NGRP = NCHUNKS // GROUP
EROWS = 2 * E // EDGE_CHUNK
DROWS = 16
DCHUNKS = NCHUNKS // DROWS


def _mesh():
    return plsc.VectorSubcoreMesh(
        core_axis_name="c", subcore_axis_name="s",
        num_cores=NC, num_subcores=NS)


DTILE_ROWS = 2 * NCHUNKS
DNCHUNK = DTILE_ROWS // DROWS


def _degree_body(edge_hbm, dinv_hbm, idx_v, deg_v, acc_v, part0_v, part1_v,
                 dv_v, shared, semi0, semi1):
    c = lax.axis_index("c")
    s = lax.axis_index("s")
    dst_base = EROWS // 2 + s * DTILE_ROWS
    zero16 = jnp.zeros((L,), jnp.float32)
    ones16 = jnp.ones((L,), jnp.float32)
    tail_mask = lax.iota(jnp.int32, L) >= 3

    def zero_body(i, carry):
        for u in range(8):
            deg_v[pl.ds((i * 8 + u) * L, L)] = zero16
        return carry
    lax.fori_loop(0, NP // L // 8, zero_body, 0)

    semi = (semi0, semi1)
    pend = [None, None]
    pend[0] = pltpu.async_copy(
        edge_hbm.at[pl.ds(dst_base, DROWS)], idx_v.at[0], semi[0])
    for k in range(DNCHUNK):
        b = k % 2
        if k + 1 < DNCHUNK:
            pend[1 - b] = pltpu.async_copy(
                edge_hbm.at[pl.ds(dst_base + (k + 1) * DROWS, DROWS)],
                idx_v.at[1 - b], semi[1 - b])
        pend[b].wait()
        for r in range(DROWS):
            for q in range(7):
                idx16 = idx_v[b, r, pl.ds(q * L, L)]
                plsc.addupdate_scatter(deg_v, [idx16], ones16)
            idx16 = idx_v[b, r, pl.ds(109, L)]
            plsc.addupdate_scatter(deg_v, [idx16], ones16, mask=tail_mask)

    pltpu.sync_copy(deg_v, shared.at[s])
    plsc.subcore_barrier()
    pltpu.sync_copy(shared.at[0, pl.ds(s * ROWS_PER_TILE, ROWS_PER_TILE)], acc_v)
    parts = (part0_v, part1_v)
    pendc = [None, None]
    pendc[1] = pltpu.async_copy(
        shared.at[1, pl.ds(s * ROWS_PER_TILE, ROWS_PER_TILE)], parts[1],
        semi0)
    for p in range(1, NS):
        pb = p % 2
        if p + 1 < NS:
            pendc[1 - pb] = pltpu.async_copy(
                shared.at[p + 1, pl.ds(s * ROWS_PER_TILE, ROWS_PER_TILE)],
                parts[1 - pb], semi1 if pb else semi0)
        pendc[pb].wait()

        def add_body(i, carry):
            for u in range(4):
                o = (i * 4 + u) * L
                acc_v[pl.ds(o, L)] = (acc_v[pl.ds(o, L)]
                                      + parts[pb][pl.ds(o, L)])
            return carry
        lax.fori_loop(0, ROWS_PER_TILE // L // 4, add_body, 0)

    def rsqrt_body(i, carry):
        d = acc_v[pl.ds(i * L, L)] + 1.0
        bits = plsc.bitcast(d, jnp.int32)
        bits = 0x5F3759DF - (bits >> 1)
        y = plsc.bitcast(bits, jnp.float32)
        for _ in range(3):
            y = y * (1.5 - 0.5 * d * y * y)
        dv_v[pl.ds(i * L, L)] = y
        return carry
    lax.fori_loop(0, ROWS_PER_TILE // L, rsqrt_body, 0)

    @pl.when(c == 0)
    def _():
        pltpu.sync_copy(
            dv_v, dinv_hbm.at[pl.ds(s * ROWS_PER_TILE, ROWS_PER_TILE)])


@functools.cache
def _degree_kernel_fn():
    return pl.kernel(
        _degree_body,
        out_type=jax.ShapeDtypeStruct((NP,), jnp.float32),
        mesh=_mesh(),
        compiler_params=pltpu.CompilerParams(needs_layout_passes=False),
        scratch_types=[
            pltpu.VMEM((2, DROWS, EDGE_CHUNK), jnp.int32),
            pltpu.VMEM((NP,), jnp.float32),
            pltpu.VMEM((ROWS_PER_TILE,), jnp.float32),
            pltpu.VMEM((ROWS_PER_TILE,), jnp.float32),
            pltpu.VMEM((ROWS_PER_TILE,), jnp.float32),
            pltpu.VMEM((ROWS_PER_TILE,), jnp.float32),
            pltpu.VMEM_SHARED((NS, NP), jnp.float32),
            pltpu.SemaphoreType.DMA,
            pltpu.SemaphoreType.DMA,
        ],
    )


def _degree_kernel(edge2d):
    return _degree_kernel_fn()(edge2d)


def _aggregate_body(edge_hbm, g_hbm, acc_hbm,
                    src_v, dst_v, rows0, rows1, acc_sh,
                    semg0, semg1, sems0, sems1, semi0, semi1, semi2):
    c = lax.axis_index("c")
    s = lax.axis_index("s")
    w = s * NC + c
    base = w * NCHUNKS

    pltpu.sync_copy(g_hbm.at[pl.ds(s * ROWS_PER_TILE, ROWS_PER_TILE)],
                    acc_sh.at[pl.ds(s * ROWS_PER_TILE, ROWS_PER_TILE)])

    pltpu.sync_copy(edge_hbm.at[pl.ds(base, GROUP)], src_v.at[0])
    pltpu.sync_copy(edge_hbm.at[pl.ds(EROWS // 2 + base, GROUP)], dst_v.at[0])
    plsc.subcore_barrier()

    rows = (rows0, rows1)
    semg = (semg0, semg1)
    sems = (sems0, sems1)
    semi = (semi0, semi1, semi2)
    pendg = [None, None]
    pends = [None, None]
    pendi = [None, None]
    pendg[0] = pltpu.async_copy(g_hbm.at[src_v.at[0, 0]], rows[0], semg[0])
    for grp in range(NGRP):
        gb = grp % 3
        if grp + 1 < NGRP:
            nb = (grp + 1) % 3
            nbase = base + (grp + 1) * GROUP
            pendi[0] = pltpu.async_copy(
                edge_hbm.at[pl.ds(nbase, GROUP)], src_v.at[nb], semi[nb])
            pendi[1] = pltpu.async_copy(
                edge_hbm.at[pl.ds(EROWS // 2 + nbase, GROUP)], dst_v.at[nb],
                semi[nb])
        for j in range(GROUP):
            k = grp * GROUP + j
            b = k % 2
            if k + 1 < NCHUNKS:
                g2 = (k + 1) // GROUP
                if g2 != grp:
                    pendi[0].wait()
                    pendi[1].wait()
                if pends[1 - b] is not None:
                    pends[1 - b].wait()
                    pends[1 - b] = None
                pendg[1 - b] = pltpu.async_copy(
                    g_hbm.at[src_v.at[g2 % 3, (k + 1) % GROUP]],
                    rows[1 - b], semg[1 - b])
            pendg[b].wait()
            pends[b] = pltpu.async_copy(
                rows[b], acc_sh.at[dst_v.at[gb, j]], sems[b], add=True)
    for b in range(2):
        if pends[b] is not None:
            pends[b].wait()

    plsc.subcore_barrier()
    pltpu.sync_copy(
        acc_sh.at[pl.ds(s * ROWS_PER_TILE, ROWS_PER_TILE)],
        acc_hbm.at[pl.ds(c * NP + s * ROWS_PER_TILE, ROWS_PER_TILE)])


@functools.cache
def _aggregate_kernel_fn():
    return pl.kernel(
        _aggregate_body,
        out_type=jax.ShapeDtypeStruct((NC * NP, D), jnp.float32),
        mesh=_mesh(),
        compiler_params=pltpu.CompilerParams(needs_layout_passes=False),
        scratch_types=[
            pltpu.VMEM((3, GROUP, EDGE_CHUNK), jnp.int32),
            pltpu.VMEM((3, GROUP, EDGE_CHUNK), jnp.int32),
            pltpu.VMEM((EDGE_CHUNK, D), jnp.float32),
            pltpu.VMEM((EDGE_CHUNK, D), jnp.float32),
            pltpu.VMEM_SHARED((NP, D), jnp.float32),
            pltpu.SemaphoreType.DMA,
            pltpu.SemaphoreType.DMA,
            pltpu.SemaphoreType.DMA,
            pltpu.SemaphoreType.DMA,
            pltpu.SemaphoreType.DMA,
            pltpu.SemaphoreType.DMA,
            pltpu.SemaphoreType.DMA,
        ],
    )


def _aggregate_kernel(edge2d, g):
    return _aggregate_kernel_fn()(edge2d, g)


RB = 1024
GRID = NP // RB


def _mlp_body(x_ref, wi_ref, bi_ref, wh_ref, bh_ref, wg_ref,
              feat_ref, h_ref):
    t = jnp.dot(x_ref[...], wi_ref[...], preferred_element_type=jnp.float32)
    t = jnp.maximum(t + bi_ref[...], 0.0)
    f = jnp.dot(t, wh_ref[...], preferred_element_type=jnp.float32)
    f = jnp.maximum(f + bh_ref[...], 0.0)
    feat_ref[...] = f
    h_ref[...] = jnp.dot(f, wg_ref[...], preferred_element_type=jnp.float32)


def _run_mlp(x, w_in, b_in, w_h, b_h, w_g):
    wspec = pl.BlockSpec((D, D), lambda i: (0, 0))
    bspec = pl.BlockSpec((1, D), lambda i: (0, 0))
    return pl.pallas_call(
        _mlp_body,
        grid=(GRID,),
        in_specs=[
            pl.BlockSpec((RB, D), lambda i: (i, 0)),
            wspec, bspec, wspec, bspec, wspec,
        ],
        out_specs=[pl.BlockSpec((RB, D), lambda i: (i, 0)),
                   pl.BlockSpec((RB, D), lambda i: (i, 0))],
        out_shape=[jax.ShapeDtypeStruct((N, D), jnp.float32),
                   jax.ShapeDtypeStruct((NP, D), jnp.float32)],
    )(x, w_in, b_in, w_h, b_h, w_g)


def _scale_body(h_ref, dinv_ref, g_ref):
    g_ref[...] = h_ref[...] * dinv_ref[...]


def _run_scale(h, dinv2d):
    return pl.pallas_call(
        _scale_body,
        grid=(GRID,),
        in_specs=[pl.BlockSpec((RB, D), lambda i: (i, 0)),
                  pl.BlockSpec((RB, 1), lambda i: (i, 0))],
        out_specs=pl.BlockSpec((RB, D), lambda i: (i, 0)),
        out_shape=jax.ShapeDtypeStruct((NP, D), jnp.float32),
    )(h, dinv2d)


def _final_body(acc0_ref, acc1_ref, g_ref, dinv_ref, bg_ref, out_ref):
    total = acc0_ref[...] + acc1_ref[...] - g_ref[...]
    out_ref[...] = total * dinv_ref[...] + bg_ref[...]


def _run_final(acc_flat, g, dinv2d, b_g):
    return pl.pallas_call(
        _final_body,
        grid=(GRID,),
        in_specs=[
            pl.BlockSpec((RB, D), lambda i: (i, 0)),
            pl.BlockSpec((RB, D), lambda i: (GRID + i, 0)),
            pl.BlockSpec((RB, D), lambda i: (i, 0)),
            pl.BlockSpec((RB, 1), lambda i: (i, 0)),
            pl.BlockSpec((1, D), lambda i: (0, 0)),
        ],
        out_specs=pl.BlockSpec((RB, D), lambda i: (i, 0)),
        out_shape=jax.ShapeDtypeStruct((N, D), jnp.float32),
    )(acc_flat, acc_flat, g, dinv2d, b_g)


def kernel(x, edge_index, W_in, b_in, W_h, b_h, W_g, b_g):
    edge_index = edge_index.astype(jnp.int32)
    edge2d = edge_index.reshape(EROWS, EDGE_CHUNK)

    feat, h = _run_mlp(x, W_in, b_in.reshape(1, D), W_h, b_h.reshape(1, D), W_g)
    dinv2d = _degree_kernel(edge2d).reshape(NP, 1)
    g = _run_scale(h, dinv2d)

    acc_flat = _aggregate_kernel(edge2d, g)

    out_feat = _run_final(acc_flat, g, dinv2d, b_g.reshape(1, D))
    return (feat, out_feat)
